# baseline (speedup 1.0000x reference)
import functools

import jax
import jax.numpy as jnp
from jax import lax
from jax.experimental import pallas as pl
from jax.experimental.pallas import tpu as pltpu
from jax.experimental.pallas import tpu_sc as plsc

_D = 768
_H = 12
_DH = 64
_E = 32
_FF = 2048
_B = 16
_S = 65
_T = _B * _S
_C = 65
_CP = 72
_NSLOT = _E * _CP
_TP = 1280
_NACT = 1968
_R = 208
_NBLK = _T // _R

_HI = lax.Precision.HIGHEST


def _dot(a, b, prec=_HI):
    return jax.lax.dot(a, b, preferred_element_type=jnp.float32, precision=prec)


def _bdot(a, b):
    return jax.lax.dot(a.astype(jnp.bfloat16), b.astype(jnp.bfloat16),
                       preferred_element_type=jnp.float32)


def _b16(x):
    return x.astype(jnp.bfloat16).astype(jnp.float32)


def _rmsn(x, w):
    return x * lax.rsqrt(jnp.mean(x * x, axis=-1, keepdims=True) + 1e-6) * w


def _k_embed_qkv(i1_ref, i2_ref, pe_ref, ce_ref, ln_ref, wq_ref, wk_ref,
                 wv_ref, x_ref, q_ref, k_ref, v_ref):
    i1 = i1_ref[...]
    i2 = i2_ref[...]
    oh1 = (lax.broadcasted_iota(jnp.int32, (_R, 65), 1) == i1).astype(jnp.float32)
    oh2 = (lax.broadcasted_iota(jnp.int32, (_R, 14), 1) == i2).astype(jnp.float32)
    x = _dot(oh1, pe_ref[...]) + _dot(oh2, ce_ref[...])
    x_ref[...] = x
    h = _rmsn(x, ln_ref[...])
    q_ref[...] = _bdot(h, wq_ref[...])
    k_ref[...] = _bdot(h, wk_ref[...])
    v_ref[...] = _bdot(h, wv_ref[...])


def _k_attn(q_ref, k_ref, v_ref, xres_ref, wo_ref, out_ref):
    h = pl.program_id(1)
    q = q_ref[0, 0]
    k = k_ref[0, 0]
    v = v_ref[0, 0]
    s = lax.dot_general(q.astype(jnp.bfloat16), k.astype(jnp.bfloat16),
                        (((1,), (1,)), ((), ())),
                        preferred_element_type=jnp.float32) * (1.0 / 8.0)
    m = jnp.max(s, axis=-1, keepdims=True)
    p = jnp.exp(s - m)
    p = p / jnp.sum(p, axis=-1, keepdims=True)
    o = _bdot(p, v)
    contrib = _bdot(o, wo_ref[...])

    @pl.when(h == 0)
    def _():
        out_ref[...] = xres_ref[...] + contrib[None]

    @pl.when(h > 0)
    def _():
        out_ref[...] += contrib[None]


def _k_ffn_qkv(x_ref, ln2_ref, wi_ref, wo_ref, ln1_ref, wq_ref, wk_ref,
               wv_ref, x2_ref, q_ref, k_ref, v_ref):
    x = x_ref[...]
    h = _rmsn(x, ln2_ref[...])
    f = jnp.maximum(_bdot(h, wi_ref[...]), 0.0)
    x2 = x + _bdot(f, wo_ref[...])
    x2_ref[...] = x2
    h2 = _rmsn(x2, ln1_ref[...])
    q_ref[...] = _bdot(h2, wq_ref[...])
    k_ref[...] = _bdot(h2, wk_ref[...])
    v_ref[...] = _bdot(h2, wv_ref[...])


def _k_router(x_ref, ln_ref, rw_ref, h_ref, probs_ref, gate_ref, slot_ref,
              src8_ref, carry_ref):
    i = pl.program_id(0)

    @pl.when(i == 0)
    def _():
        carry_ref[...] = jnp.zeros_like(carry_ref)

    x = x_ref[...]
    h = _rmsn(x, ln_ref[...])
    h_ref[...] = h
    logits = _bdot(h, rw_ref[...])
    m = jnp.max(logits, axis=-1, keepdims=True)
    ex = jnp.exp(logits - m)
    probs = ex / jnp.sum(ex, axis=-1, keepdims=True)
    probs_ref[...] = probs
    gate = jnp.max(probs, axis=-1, keepdims=True)
    eiota = lax.broadcasted_iota(jnp.int32, (_R, _E), 1)
    idx = jnp.min(jnp.where(probs == gate, eiota, _E), axis=-1, keepdims=True)
    oh = (eiota == idx).astype(jnp.float32)
    tril = (lax.broadcasted_iota(jnp.int32, (_R, _R), 1)
            <= lax.broadcasted_iota(jnp.int32, (_R, _R), 0)).astype(jnp.float32)
    base = carry_ref[...]
    pos = _dot(tril, oh) + base
    carry_ref[...] = base + jnp.sum(oh, axis=0, keepdims=True)
    pos_tok = jnp.sum(pos * oh, axis=-1, keepdims=True)
    keep = pos_tok <= float(_C)
    slot = idx * _CP + (pos_tok.astype(jnp.int32) - 1)
    slot_ref[...] = jnp.where(keep, slot, 0)
    gate_ref[...] = jnp.where(keep, gate, 0.0)
    sdisp = jnp.where(keep, slot, _NSLOT + 7)
    ohts = (lax.broadcasted_iota(jnp.int32, (_R, _NSLOT), 1)
            == sdisp).astype(jnp.float32)
    tvals = (lax.broadcasted_iota(jnp.int32, (_R, 8), 0)
             + i * _R).astype(jnp.float32)
    contrib = lax.dot_general(ohts, tvals, (((0,), (0,)), ((), ())),
                              preferred_element_type=jnp.float32,
                              precision=_HI)

    @pl.when(i == 0)
    def _():
        src8_ref[...] = contrib

    @pl.when(i > 0)
    def _():
        src8_ref[...] += contrib


def _k_expert(b_ref, wi_ref, wo_ref, o_ref):
    x = b_ref[0]
    hmid = jnp.maximum(_bdot(x, wi_ref[0]), 0.0)
    o_ref[0] = _bdot(hmid, wo_ref[0])


def _k_final(x_ref, y_ref, gate_ref, ln_ref, w3_ref, b3_ref, enc_ref, o3_ref):
    xd = x_ref[...] + _b16(gate_ref[...]) * _b16(y_ref[...])
    enc = _rmsn(xd, ln_ref[...])
    enc_ref[...] = enc
    o3_ref[...] = _bdot(enc, w3_ref[...]) + b3_ref[...]


def _k_out1(enc_ref, w1_ref, mask_ref, b1_ref, rw_ref, rb_ref, out_ref, val_ref):
    s = pl.program_id(0)
    e = enc_ref[0]
    contrib = _bdot(e, w1_ref[0])
    vcontrib = jnp.sum(e * rw_ref[0], axis=-1, keepdims=True)

    @pl.when(s == 0)
    def _():
        out_ref[...] = contrib
        val_ref[...] = vcontrib

    @pl.when(s > 0)
    def _():
        out_ref[...] += contrib
        val_ref[...] += vcontrib

    @pl.when(s == _S - 1)
    def _():
        o = out_ref[...] + b1_ref[...]
        out_ref[...] = jnp.where(mask_ref[...] < -1.0, -1e30, o)
        val_ref[...] = jnp.tanh(val_ref[...] + rb_ref[...])


def _row_gather(table, idx):
    v, d = table.shape
    n = idx.shape[0]
    info = plsc.get_sparse_core_info()
    nw = info.num_cores * info.num_subcores
    b_per_w = n // nw
    mesh = plsc.VectorSubcoreMesh(core_axis_name="c", subcore_axis_name="s")

    @functools.partial(
        pl.kernel, mesh=mesh,
        out_type=jax.ShapeDtypeStruct((n, d), jnp.float32),
        scratch_types=[
            pltpu.VMEM((b_per_w,), jnp.int32),
            pltpu.VMEM((b_per_w, d), jnp.float32),
            pltpu.SemaphoreType.DMA,
        ],
    )
    def gk(table_hbm, idx_hbm, out_hbm, idx_v, rows_v, sem):
        wid = lax.axis_index("s") * info.num_cores + lax.axis_index("c")
        base = wid * b_per_w
        pltpu.sync_copy(idx_hbm.at[pl.ds(base, b_per_w)], idx_v)
        pltpu.async_copy(table_hbm.at[idx_v], rows_v, sem).wait()
        pltpu.sync_copy(rows_v, out_hbm.at[pl.ds(base, b_per_w)])

    return gk(table, idx)


def kernel(input1, input2, mask, pos_emb, piece_emb, ln1_0, wq_0, wk_0, wv_0,
           wo_0, ln2_0, ffn_wi_0, ffn_wo_0, ln1_1, wq_1, wk_1, wv_1, wo_1,
           ln2_1, router_w, exp_wi, exp_wo, final_ln, out1_w, out1_b, out3_w,
           out3_b, rew_w, rew_b):
    f32 = jnp.float32
    i1 = input1.reshape(_T, 1).astype(jnp.int32)
    i2 = input2.reshape(_T, 1).astype(jnp.int32)

    rowspec = pl.BlockSpec((_R, _D), lambda i: (i, 0))
    col1spec = pl.BlockSpec((_R, 1), lambda i: (i, 0))

    def const(shape):
        nd = len(shape)
        return pl.BlockSpec(shape, lambda i, _n=nd: (0,) * _n)

    x0, q0, k0, v0 = pl.pallas_call(
        _k_embed_qkv,
        grid=(_NBLK,),
        in_specs=[col1spec, col1spec, const((65, _D)), const((14, _D)),
                  const((1, _D)), const((_D, _D)), const((_D, _D)),
                  const((_D, _D))],
        out_specs=[rowspec] * 4,
        out_shape=[jax.ShapeDtypeStruct((_T, _D), f32)] * 4,
    )(i1, i2, pos_emb, piece_emb, ln1_0.reshape(1, _D), wq_0, wk_0, wv_0)

    def attn(q, k, v, xres, wo):
        q4 = q.reshape(_B, _S, _H, _DH).transpose(0, 2, 1, 3)
        k4 = k.reshape(_B, _S, _H, _DH).transpose(0, 2, 1, 3)
        v4 = v.reshape(_B, _S, _H, _DH).transpose(0, 2, 1, 3)
        hspec = pl.BlockSpec((1, 1, _S, _DH), lambda b, h: (b, h, 0, 0))
        return pl.pallas_call(
            _k_attn,
            grid=(_B, _H),
            in_specs=[hspec, hspec, hspec,
                      pl.BlockSpec((1, _S, _D), lambda b, h: (b, 0, 0)),
                      pl.BlockSpec((_DH, _D), lambda b, h: (h, 0))],
            out_specs=pl.BlockSpec((1, _S, _D), lambda b, h: (b, 0, 0)),
            out_shape=jax.ShapeDtypeStruct((_B, _S, _D), f32),
        )(q4, k4, v4, xres.reshape(_B, _S, _D), wo)

    xa = attn(q0, k0, v0, x0, wo_0).reshape(_T, _D)

    xb, q1, k1, v1 = pl.pallas_call(
        _k_ffn_qkv,
        grid=(_NBLK,),
        in_specs=[rowspec, const((1, _D)), const((_D, _FF)), const((_FF, _D)),
                  const((1, _D)), const((_D, _D)), const((_D, _D)),
                  const((_D, _D))],
        out_specs=[rowspec] * 4,
        out_shape=[jax.ShapeDtypeStruct((_T, _D), f32)] * 4,
    )(xa, ln2_0.reshape(1, _D), ffn_wi_0, ffn_wo_0, ln1_1.reshape(1, _D),
      wq_1, wk_1, wv_1)

    xc = attn(q1, k1, v1, xb, wo_1).reshape(_T, _D)

    h, probs, gate, slot, src8 = pl.pallas_call(
        _k_router,
        grid=(_NBLK,),
        in_specs=[rowspec, const((1, _D)), const((_D, _E))],
        out_specs=[rowspec, pl.BlockSpec((_R, _E), lambda i: (i, 0)),
                   col1spec, col1spec, const((_NSLOT, 8))],
        out_shape=[
            jax.ShapeDtypeStruct((_T, _D), f32),
            jax.ShapeDtypeStruct((_T, _E), f32),
            jax.ShapeDtypeStruct((_T, 1), f32),
            jax.ShapeDtypeStruct((_T, 1), jnp.int32),
            jax.ShapeDtypeStruct((_NSLOT, 8), f32),
        ],
        scratch_shapes=[pltpu.VMEM((1, _E), f32)],
    )(xc, ln2_1.reshape(1, _D), router_w)

    src = src8[:, 0].astype(jnp.int32)
    buf = _row_gather(h, src)

    eo = pl.pallas_call(
        _k_expert,
        grid=(_E,),
        in_specs=[pl.BlockSpec((1, _CP, _D), lambda e: (e, 0, 0)),
                  pl.BlockSpec((1, _D, _FF), lambda e: (e, 0, 0)),
                  pl.BlockSpec((1, _FF, _D), lambda e: (e, 0, 0))],
        out_specs=pl.BlockSpec((1, _CP, _D), lambda e: (e, 0, 0)),
        out_shape=jax.ShapeDtypeStruct((_E, _CP, _D), f32),
    )(buf.reshape(_E, _CP, _D), exp_wi, exp_wo)

    slot_pad = jnp.concatenate(
        [slot.reshape(_T), jnp.zeros((_TP - _T,), jnp.int32)])
    y = _row_gather(eo.reshape(_NSLOT, _D), slot_pad)[:_T]

    encoded, out3 = pl.pallas_call(
        _k_final,
        grid=(_NBLK,),
        in_specs=[rowspec, rowspec, col1spec, const((1, _D)),
                  const((_D, 14)), const((1, 14))],
        out_specs=[rowspec, pl.BlockSpec((_R, 14), lambda i: (i, 0))],
        out_shape=[jax.ShapeDtypeStruct((_T, _D), f32),
                   jax.ShapeDtypeStruct((_T, 14), f32)],
    )(xc, y, gate, final_ln.reshape(1, _D), out3_w, out3_b.reshape(1, 14))

    enc_t = encoded.reshape(_B, _S, _D).transpose(1, 0, 2)
    w1r = out1_w.reshape(_S, _D, _NACT)
    rwr = rew_w.reshape(_S, 1, _D)
    out1_final, values = pl.pallas_call(
        _k_out1,
        grid=(_S,),
        in_specs=[pl.BlockSpec((1, _B, _D), lambda s: (s, 0, 0)),
                  pl.BlockSpec((1, _D, _NACT), lambda s: (s, 0, 0)),
                  pl.BlockSpec((_B, _NACT), lambda s: (0, 0)),
                  pl.BlockSpec((1, _NACT), lambda s: (0, 0)),
                  pl.BlockSpec((1, 1, _D), lambda s: (s, 0, 0)),
                  pl.BlockSpec((1, 1), lambda s: (0, 0))],
        out_specs=[pl.BlockSpec((_B, _NACT), lambda s: (0, 0)),
                   pl.BlockSpec((_B, 1), lambda s: (0, 0))],
        out_shape=[jax.ShapeDtypeStruct((_B, _NACT), f32),
                   jax.ShapeDtypeStruct((_B, 1), f32)],
    )(enc_t, w1r, mask, out1_b.reshape(1, _NACT), rwr, rew_b.reshape(1, 1))

    return (out1_final, out3.reshape(_B, _S, 14),
            encoded.reshape(_B, _S, _D)[:, -1, :], values,
            probs.reshape(_B, _S, _E))


# speedup vs baseline: 1.1609x; 1.1609x over previous
import functools

import jax
import jax.numpy as jnp
from jax import lax
from jax.experimental import pallas as pl
from jax.experimental.pallas import tpu as pltpu
from jax.experimental.pallas import tpu_sc as plsc

_D = 768
_H = 12
_DH = 64
_E = 32
_FF = 2048
_B = 16
_S = 65
_T = _B * _S
_C = 65
_CP = 72
_NSLOT = _E * _CP
_TP = 1280
_NACT = 1968
_R = 208
_NBLK = _T // _R

_HI = lax.Precision.HIGHEST


def _dot(a, b, prec=_HI):
    return jax.lax.dot(a, b, preferred_element_type=jnp.float32, precision=prec)


def _bdot(a, b):
    return jax.lax.dot(a.astype(jnp.bfloat16), b.astype(jnp.bfloat16),
                       preferred_element_type=jnp.float32)


def _b16(x):
    return x.astype(jnp.bfloat16).astype(jnp.float32)


def _rmsn(x, w):
    return x * lax.rsqrt(jnp.mean(x * x, axis=-1, keepdims=True) + 1e-6) * w


def _k_embed_qkv(i1_ref, i2_ref, pe_ref, ce_ref, ln_ref, wq_ref, wk_ref,
                 wv_ref, x_ref, q_ref, k_ref, v_ref):
    i1 = i1_ref[...]
    i2 = i2_ref[...]
    oh1 = (lax.broadcasted_iota(jnp.int32, (_R, 65), 1) == i1).astype(jnp.float32)
    oh2 = (lax.broadcasted_iota(jnp.int32, (_R, 14), 1) == i2).astype(jnp.float32)
    x = _dot(oh1, pe_ref[...]) + _dot(oh2, ce_ref[...])
    x_ref[...] = x
    h = _rmsn(x, ln_ref[...])
    q_ref[...] = _bdot(h, wq_ref[...])
    k_ref[...] = _bdot(h, wk_ref[...])
    v_ref[...] = _bdot(h, wv_ref[...])


def _k_attn(q_ref, k_ref, v_ref, xres_ref, wo_ref, out_ref):
    h2 = pl.program_id(1)
    q2 = q_ref[0]
    k2 = k_ref[0]
    v2 = v_ref[0]
    outs = []
    for off in (0, _DH):
        qh = q2[:, off:off + _DH]
        kh = k2[:, off:off + _DH]
        vh = v2[:, off:off + _DH]
        s = lax.dot_general(qh.astype(jnp.bfloat16), kh.astype(jnp.bfloat16),
                            (((1,), (1,)), ((), ())),
                            preferred_element_type=jnp.float32) * (1.0 / 8.0)
        m = jnp.max(s, axis=-1, keepdims=True)
        p = jnp.exp(s - m)
        p = p / jnp.sum(p, axis=-1, keepdims=True)
        outs.append(_bdot(p, vh))
    o2 = jnp.concatenate(outs, axis=-1)
    contrib = _bdot(o2, wo_ref[...])

    @pl.when(h2 == 0)
    def _():
        out_ref[...] = xres_ref[...] + contrib[None]

    @pl.when(h2 > 0)
    def _():
        out_ref[...] += contrib[None]


def _k_ffn_qkv(x_ref, ln2_ref, wi_ref, wo_ref, ln1_ref, wq_ref, wk_ref,
               wv_ref, x2_ref, q_ref, k_ref, v_ref):
    x = x_ref[...]
    h = _rmsn(x, ln2_ref[...])
    f = jnp.maximum(_bdot(h, wi_ref[...]), 0.0)
    x2 = x + _bdot(f, wo_ref[...])
    x2_ref[...] = x2
    h2 = _rmsn(x2, ln1_ref[...])
    q_ref[...] = _bdot(h2, wq_ref[...])
    k_ref[...] = _bdot(h2, wk_ref[...])
    v_ref[...] = _bdot(h2, wv_ref[...])


def _k_router(x_ref, ln_ref, rw_ref, h_ref, probs_ref, gate_ref, slot_ref,
              src8_ref, carry_ref):
    i = pl.program_id(0)

    @pl.when(i == 0)
    def _():
        carry_ref[...] = jnp.zeros_like(carry_ref)

    x = x_ref[...]
    h = _rmsn(x, ln_ref[...])
    h_ref[...] = h
    logits = _bdot(h, rw_ref[...])
    m = jnp.max(logits, axis=-1, keepdims=True)
    ex = jnp.exp(logits - m)
    probs = ex / jnp.sum(ex, axis=-1, keepdims=True)
    probs_ref[...] = probs
    gate = jnp.max(probs, axis=-1, keepdims=True)
    eiota = lax.broadcasted_iota(jnp.int32, (_R, _E), 1)
    idx = jnp.min(jnp.where(probs == gate, eiota, _E), axis=-1, keepdims=True)
    oh = (eiota == idx).astype(jnp.float32)
    tril = (lax.broadcasted_iota(jnp.int32, (_R, _R), 1)
            <= lax.broadcasted_iota(jnp.int32, (_R, _R), 0)).astype(jnp.float32)
    base = carry_ref[...]
    pos = _dot(tril, oh) + base
    carry_ref[...] = base + jnp.sum(oh, axis=0, keepdims=True)
    pos_tok = jnp.sum(pos * oh, axis=-1, keepdims=True)
    keep = pos_tok <= float(_C)
    slot = idx * _CP + (pos_tok.astype(jnp.int32) - 1)
    slot_ref[...] = jnp.where(keep, slot, 0)
    gate_ref[...] = jnp.where(keep, gate, 0.0)
    sdisp = jnp.where(keep, slot, _NSLOT + 7)
    ohts = (lax.broadcasted_iota(jnp.int32, (_R, _NSLOT), 1)
            == sdisp).astype(jnp.float32)
    tvals = (lax.broadcasted_iota(jnp.int32, (_R, 8), 0)
             + i * _R).astype(jnp.float32)
    contrib = lax.dot_general(ohts, tvals, (((0,), (0,)), ((), ())),
                              preferred_element_type=jnp.float32,
                              precision=_HI)

    @pl.when(i == 0)
    def _():
        src8_ref[...] = contrib

    @pl.when(i > 0)
    def _():
        src8_ref[...] += contrib


def _k_expert(b_ref, wi_ref, wo_ref, o_ref):
    x = b_ref[0]
    hmid = jnp.maximum(_bdot(x, wi_ref[0]), 0.0)
    o_ref[0] = _bdot(hmid, wo_ref[0])


def _k_final(x_ref, y_ref, gate_ref, ln_ref, w3_ref, b3_ref, enc_ref, o3_ref):
    xd = x_ref[...] + _b16(gate_ref[...]) * _b16(y_ref[...])
    enc = _rmsn(xd, ln_ref[...])
    enc_ref[...] = enc
    o3_ref[...] = _bdot(enc, w3_ref[...]) + b3_ref[...]


def _k_out1(enc_ref, w1_ref, mask_ref, b1_ref, rw_ref, rb_ref, out_ref, val_ref):
    s = pl.program_id(0)
    e = enc_ref[:, 0, 0, :]
    contrib = _bdot(e, w1_ref[0])
    vcontrib = jnp.sum(e * rw_ref[0], axis=-1, keepdims=True)

    @pl.when(s == 0)
    def _():
        out_ref[...] = contrib
        val_ref[...] = vcontrib

    @pl.when(s > 0)
    def _():
        out_ref[...] += contrib
        val_ref[...] += vcontrib

    @pl.when(s == _S - 1)
    def _():
        o = out_ref[...] + b1_ref[...]
        out_ref[...] = jnp.where(mask_ref[...] < -1.0, -1e30, o)
        val_ref[...] = jnp.tanh(val_ref[...] + rb_ref[...])


def _row_gather(table, idx):
    v, d = table.shape
    n = idx.shape[0]
    info = plsc.get_sparse_core_info()
    nw = info.num_cores * info.num_subcores
    b_per_w = n // nw
    mesh = plsc.VectorSubcoreMesh(core_axis_name="c", subcore_axis_name="s")

    @functools.partial(
        pl.kernel, mesh=mesh,
        out_type=jax.ShapeDtypeStruct((n, d), jnp.float32),
        scratch_types=[
            pltpu.VMEM((b_per_w,), jnp.int32),
            pltpu.VMEM((b_per_w, d), jnp.float32),
            pltpu.SemaphoreType.DMA,
        ],
    )
    def gk(table_hbm, idx_hbm, out_hbm, idx_v, rows_v, sem):
        wid = lax.axis_index("s") * info.num_cores + lax.axis_index("c")
        base = wid * b_per_w
        pltpu.sync_copy(idx_hbm.at[pl.ds(base, b_per_w)], idx_v)
        pltpu.async_copy(table_hbm.at[idx_v], rows_v, sem).wait()
        pltpu.sync_copy(rows_v, out_hbm.at[pl.ds(base, b_per_w)])

    return gk(table, idx)


def kernel(input1, input2, mask, pos_emb, piece_emb, ln1_0, wq_0, wk_0, wv_0,
           wo_0, ln2_0, ffn_wi_0, ffn_wo_0, ln1_1, wq_1, wk_1, wv_1, wo_1,
           ln2_1, router_w, exp_wi, exp_wo, final_ln, out1_w, out1_b, out3_w,
           out3_b, rew_w, rew_b):
    f32 = jnp.float32
    i1 = input1.reshape(_T, 1).astype(jnp.int32)
    i2 = input2.reshape(_T, 1).astype(jnp.int32)

    rowspec = pl.BlockSpec((_R, _D), lambda i: (i, 0))
    col1spec = pl.BlockSpec((_R, 1), lambda i: (i, 0))

    def const(shape):
        nd = len(shape)
        return pl.BlockSpec(shape, lambda i, _n=nd: (0,) * _n)

    x0, q0, k0, v0 = pl.pallas_call(
        _k_embed_qkv,
        grid=(_NBLK,),
        in_specs=[col1spec, col1spec, const((65, _D)), const((14, _D)),
                  const((1, _D)), const((_D, _D)), const((_D, _D)),
                  const((_D, _D))],
        out_specs=[rowspec] * 4,
        out_shape=[jax.ShapeDtypeStruct((_T, _D), f32)] * 4,
    )(i1, i2, pos_emb, piece_emb, ln1_0.reshape(1, _D), wq_0, wk_0, wv_0)

    def attn(q, k, v, xres, wo):
        q3 = q.reshape(_B, _S, _D)
        k3 = k.reshape(_B, _S, _D)
        v3 = v.reshape(_B, _S, _D)
        hspec = pl.BlockSpec((1, _S, 128), lambda b, h2: (b, 0, h2))
        return pl.pallas_call(
            _k_attn,
            grid=(_B, _H // 2),
            in_specs=[hspec, hspec, hspec,
                      pl.BlockSpec((1, _S, _D), lambda b, h2: (b, 0, 0)),
                      pl.BlockSpec((128, _D), lambda b, h2: (h2, 0))],
            out_specs=pl.BlockSpec((1, _S, _D), lambda b, h2: (b, 0, 0)),
            out_shape=jax.ShapeDtypeStruct((_B, _S, _D), f32),
        )(q3, k3, v3, xres.reshape(_B, _S, _D), wo)

    xa = attn(q0, k0, v0, x0, wo_0).reshape(_T, _D)

    xb, q1, k1, v1 = pl.pallas_call(
        _k_ffn_qkv,
        grid=(_NBLK,),
        in_specs=[rowspec, const((1, _D)), const((_D, _FF)), const((_FF, _D)),
                  const((1, _D)), const((_D, _D)), const((_D, _D)),
                  const((_D, _D))],
        out_specs=[rowspec] * 4,
        out_shape=[jax.ShapeDtypeStruct((_T, _D), f32)] * 4,
    )(xa, ln2_0.reshape(1, _D), ffn_wi_0, ffn_wo_0, ln1_1.reshape(1, _D),
      wq_1, wk_1, wv_1)

    xc = attn(q1, k1, v1, xb, wo_1).reshape(_T, _D)

    h, probs, gate, slot, src8 = pl.pallas_call(
        _k_router,
        grid=(_NBLK,),
        in_specs=[rowspec, const((1, _D)), const((_D, _E))],
        out_specs=[rowspec, pl.BlockSpec((_R, _E), lambda i: (i, 0)),
                   col1spec, col1spec, const((_NSLOT, 8))],
        out_shape=[
            jax.ShapeDtypeStruct((_T, _D), f32),
            jax.ShapeDtypeStruct((_T, _E), f32),
            jax.ShapeDtypeStruct((_T, 1), f32),
            jax.ShapeDtypeStruct((_T, 1), jnp.int32),
            jax.ShapeDtypeStruct((_NSLOT, 8), f32),
        ],
        scratch_shapes=[pltpu.VMEM((1, _E), f32)],
    )(xc, ln2_1.reshape(1, _D), router_w)

    src = src8[:, 0].astype(jnp.int32)
    buf = _row_gather(h, src)

    eo = pl.pallas_call(
        _k_expert,
        grid=(_E,),
        in_specs=[pl.BlockSpec((1, _CP, _D), lambda e: (e, 0, 0)),
                  pl.BlockSpec((1, _D, _FF), lambda e: (e, 0, 0)),
                  pl.BlockSpec((1, _FF, _D), lambda e: (e, 0, 0))],
        out_specs=pl.BlockSpec((1, _CP, _D), lambda e: (e, 0, 0)),
        out_shape=jax.ShapeDtypeStruct((_E, _CP, _D), f32),
    )(buf.reshape(_E, _CP, _D), exp_wi, exp_wo)

    slot_pad = jnp.concatenate(
        [slot.reshape(_T), jnp.zeros((_TP - _T,), jnp.int32)])
    y = _row_gather(eo.reshape(_NSLOT, _D), slot_pad)[:_T]

    encoded, out3 = pl.pallas_call(
        _k_final,
        grid=(_NBLK,),
        in_specs=[rowspec, rowspec, col1spec, const((1, _D)),
                  const((_D, 14)), const((1, 14))],
        out_specs=[rowspec, pl.BlockSpec((_R, 14), lambda i: (i, 0))],
        out_shape=[jax.ShapeDtypeStruct((_T, _D), f32),
                   jax.ShapeDtypeStruct((_T, 14), f32)],
    )(xc, y, gate, final_ln.reshape(1, _D), out3_w, out3_b.reshape(1, 14))

    enc4 = encoded.reshape(_B, _S, 1, _D)
    w1r = out1_w.reshape(_S, _D, _NACT)
    rwr = rew_w.reshape(_S, 1, _D)
    out1_final, values = pl.pallas_call(
        _k_out1,
        grid=(_S,),
        in_specs=[pl.BlockSpec((_B, 1, 1, _D), lambda s: (0, s, 0, 0)),
                  pl.BlockSpec((1, _D, _NACT), lambda s: (s, 0, 0)),
                  pl.BlockSpec((_B, _NACT), lambda s: (0, 0)),
                  pl.BlockSpec((1, _NACT), lambda s: (0, 0)),
                  pl.BlockSpec((1, 1, _D), lambda s: (s, 0, 0)),
                  pl.BlockSpec((1, 1), lambda s: (0, 0))],
        out_specs=[pl.BlockSpec((_B, _NACT), lambda s: (0, 0)),
                   pl.BlockSpec((_B, 1), lambda s: (0, 0))],
        out_shape=[jax.ShapeDtypeStruct((_B, _NACT), f32),
                   jax.ShapeDtypeStruct((_B, 1), f32)],
    )(enc4, w1r, mask, out1_b.reshape(1, _NACT), rwr, rew_b.reshape(1, 1))

    return (out1_final, out3.reshape(_B, _S, 14),
            encoded.reshape(_B, _S, _D)[:, -1, :], values,
            probs.reshape(_B, _S, _E))


# speedup vs baseline: 1.1881x; 1.0235x over previous
import functools

import jax
import jax.numpy as jnp
from jax import lax
from jax.experimental import pallas as pl
from jax.experimental.pallas import tpu as pltpu
from jax.experimental.pallas import tpu_sc as plsc

_D = 768
_H = 12
_DH = 64
_E = 32
_FF = 2048
_B = 16
_S = 65
_SP = 72
_T = _B * _SP
_C = 65
_CP = 72
_NSLOT = _E * _CP
_TP = 1280
_NACT = 1968
_R = 192
_NBLK = _T // _R

_HI = lax.Precision.HIGHEST


def _dot(a, b, prec=_HI):
    return jax.lax.dot(a, b, preferred_element_type=jnp.float32, precision=prec)


def _bdot(a, b):
    return jax.lax.dot(a.astype(jnp.bfloat16), b.astype(jnp.bfloat16),
                       preferred_element_type=jnp.float32)


def _b16(x):
    return x.astype(jnp.bfloat16).astype(jnp.float32)


def _rmsn(x, w):
    return x * lax.rsqrt(jnp.mean(x * x, axis=-1, keepdims=True) + 1e-6) * w


def _k_embed_qkv(i1_ref, i2_ref, pe_ref, ce_ref, ln_ref, wq_ref, wk_ref,
                 wv_ref, x_ref, q_ref, k_ref, v_ref):
    i1 = i1_ref[...]
    i2 = i2_ref[...]
    oh1 = (lax.broadcasted_iota(jnp.int32, (_R, 65), 1) == i1).astype(jnp.float32)
    oh2 = (lax.broadcasted_iota(jnp.int32, (_R, 14), 1) == i2).astype(jnp.float32)
    x = _dot(oh1, pe_ref[...]) + _dot(oh2, ce_ref[...])
    x_ref[...] = x
    h = _rmsn(x, ln_ref[...])
    q_ref[...] = _bdot(h, wq_ref[...])
    k_ref[...] = _bdot(h, wk_ref[...])
    v_ref[...] = _bdot(h, wv_ref[...])


def _k_attn(q_ref, k_ref, v_ref, xres_ref, wo_ref, out_ref):
    h2 = pl.program_id(1)
    q2 = q_ref[0]
    k2 = k_ref[0]
    v2 = v_ref[0]
    kvalid = lax.broadcasted_iota(jnp.int32, (_SP, _SP), 1) < _S
    outs = []
    for off in (0, _DH):
        qh = q2[:, off:off + _DH]
        kh = k2[:, off:off + _DH]
        vh = v2[:, off:off + _DH]
        s = lax.dot_general(qh.astype(jnp.bfloat16), kh.astype(jnp.bfloat16),
                            (((1,), (1,)), ((), ())),
                            preferred_element_type=jnp.float32) * (1.0 / 8.0)
        s = jnp.where(kvalid, s, -1e30)
        m = jnp.max(s, axis=-1, keepdims=True)
        p = jnp.exp(s - m)
        p = p / jnp.sum(p, axis=-1, keepdims=True)
        outs.append(_bdot(p, vh))
    o2 = jnp.concatenate(outs, axis=-1)
    contrib = _bdot(o2, wo_ref[...])

    @pl.when(h2 == 0)
    def _():
        out_ref[...] = xres_ref[...] + contrib[None]

    @pl.when(h2 > 0)
    def _():
        out_ref[...] += contrib[None]


def _k_ffn_qkv(x_ref, ln2_ref, wi_ref, wo_ref, ln1_ref, wq_ref, wk_ref,
               wv_ref, x2_ref, q_ref, k_ref, v_ref):
    x = x_ref[...]
    h = _rmsn(x, ln2_ref[...])
    f = jnp.maximum(_bdot(h, wi_ref[...]), 0.0)
    x2 = x + _bdot(f, wo_ref[...])
    x2_ref[...] = x2
    h2 = _rmsn(x2, ln1_ref[...])
    q_ref[...] = _bdot(h2, wq_ref[...])
    k_ref[...] = _bdot(h2, wk_ref[...])
    v_ref[...] = _bdot(h2, wv_ref[...])


def _k_router(x_ref, ln_ref, rw_ref, valid_ref, h_ref, probs_ref, gate_ref,
              slot_ref, src8_ref, carry_ref):
    i = pl.program_id(0)

    @pl.when(i == 0)
    def _():
        carry_ref[...] = jnp.zeros_like(carry_ref)

    x = x_ref[...]
    h = _rmsn(x, ln_ref[...])
    h_ref[...] = h
    logits = _bdot(h, rw_ref[...])
    m = jnp.max(logits, axis=-1, keepdims=True)
    ex = jnp.exp(logits - m)
    probs = ex / jnp.sum(ex, axis=-1, keepdims=True)
    probs_ref[...] = probs
    gate = jnp.max(probs, axis=-1, keepdims=True)
    eiota = lax.broadcasted_iota(jnp.int32, (_R, _E), 1)
    idx = jnp.min(jnp.where(probs == gate, eiota, _E), axis=-1, keepdims=True)
    oh = (eiota == idx).astype(jnp.float32) * valid_ref[...]
    tril = (lax.broadcasted_iota(jnp.int32, (_R, _R), 1)
            <= lax.broadcasted_iota(jnp.int32, (_R, _R), 0)).astype(jnp.float32)
    base = carry_ref[...]
    pos = _dot(tril, oh) + base
    carry_ref[...] = base + jnp.sum(oh, axis=0, keepdims=True)
    pos_tok = jnp.sum(pos * oh, axis=-1, keepdims=True)
    keep = (pos_tok >= 1.0) & (pos_tok <= float(_C))
    slot = idx * _CP + (pos_tok.astype(jnp.int32) - 1)
    slot_ref[...] = jnp.where(keep, slot, 0)
    gate_ref[...] = jnp.where(keep, gate, 0.0)
    sdisp = jnp.where(keep, slot, _NSLOT + 7)
    ohts = (lax.broadcasted_iota(jnp.int32, (_R, _NSLOT), 1)
            == sdisp).astype(jnp.float32)
    tvals = (lax.broadcasted_iota(jnp.int32, (_R, 8), 0)
             + i * _R).astype(jnp.float32)
    contrib = lax.dot_general(ohts, tvals, (((0,), (0,)), ((), ())),
                              preferred_element_type=jnp.float32,
                              precision=_HI)

    @pl.when(i == 0)
    def _():
        src8_ref[...] = contrib

    @pl.when(i > 0)
    def _():
        src8_ref[...] += contrib


def _k_expert(b_ref, wi_ref, wo_ref, o_ref):
    x = b_ref[0]
    hmid = jnp.maximum(_bdot(x, wi_ref[0]), 0.0)
    o_ref[0] = _bdot(hmid, wo_ref[0])


def _k_final(x_ref, y_ref, gate_ref, ln_ref, w3_ref, b3_ref, enc_ref, o3_ref):
    xd = x_ref[...] + _b16(gate_ref[...]) * _b16(y_ref[...])
    enc = _rmsn(xd, ln_ref[...])
    enc_ref[...] = enc
    o3_ref[...] = _bdot(enc, w3_ref[...]) + b3_ref[...]


def _k_out1(enc_ref, w1_ref, mask_ref, b1_ref, rw_ref, rb_ref, out_ref, val_ref):
    s = pl.program_id(0)
    e = enc_ref[:, 0, 0, :]
    contrib = _bdot(e, w1_ref[0])
    vcontrib = jnp.sum(e * rw_ref[0], axis=-1, keepdims=True)

    @pl.when(s == 0)
    def _():
        out_ref[...] = contrib
        val_ref[...] = vcontrib

    @pl.when(s > 0)
    def _():
        out_ref[...] += contrib
        val_ref[...] += vcontrib

    @pl.when(s == _S - 1)
    def _():
        o = out_ref[...] + b1_ref[...]
        out_ref[...] = jnp.where(mask_ref[...] < -1.0, -1e30, o)
        val_ref[...] = jnp.tanh(val_ref[...] + rb_ref[...])


def _row_gather(table, idx):
    v, d = table.shape
    n = idx.shape[0]
    info = plsc.get_sparse_core_info()
    nw = info.num_cores * info.num_subcores
    b_per_w = n // nw
    mesh = plsc.VectorSubcoreMesh(core_axis_name="c", subcore_axis_name="s")

    @functools.partial(
        pl.kernel, mesh=mesh,
        out_type=jax.ShapeDtypeStruct((n, d), jnp.float32),
        scratch_types=[
            pltpu.VMEM((b_per_w,), jnp.int32),
            pltpu.VMEM((b_per_w, d), jnp.float32),
            pltpu.SemaphoreType.DMA,
        ],
    )
    def gk(table_hbm, idx_hbm, out_hbm, idx_v, rows_v, sem):
        wid = lax.axis_index("s") * info.num_cores + lax.axis_index("c")
        base = wid * b_per_w
        pltpu.sync_copy(idx_hbm.at[pl.ds(base, b_per_w)], idx_v)
        pltpu.async_copy(table_hbm.at[idx_v], rows_v, sem).wait()
        pltpu.sync_copy(rows_v, out_hbm.at[pl.ds(base, b_per_w)])

    return gk(table, idx)


def kernel(input1, input2, mask, pos_emb, piece_emb, ln1_0, wq_0, wk_0, wv_0,
           wo_0, ln2_0, ffn_wi_0, ffn_wo_0, ln1_1, wq_1, wk_1, wv_1, wo_1,
           ln2_1, router_w, exp_wi, exp_wo, final_ln, out1_w, out1_b, out3_w,
           out3_b, rew_w, rew_b):
    f32 = jnp.float32
    pad = ((0, 0), (0, _SP - _S))
    i1 = jnp.pad(input1, pad).reshape(_T, 1).astype(jnp.int32)
    i2 = jnp.pad(input2, pad).reshape(_T, 1).astype(jnp.int32)
    valid = (jnp.arange(_T, dtype=jnp.int32) % _SP < _S).astype(f32).reshape(_T, 1)

    rowspec = pl.BlockSpec((_R, _D), lambda i: (i, 0))
    col1spec = pl.BlockSpec((_R, 1), lambda i: (i, 0))

    def const(shape):
        nd = len(shape)
        return pl.BlockSpec(shape, lambda i, _n=nd: (0,) * _n)

    x0, q0, k0, v0 = pl.pallas_call(
        _k_embed_qkv,
        grid=(_NBLK,),
        in_specs=[col1spec, col1spec, const((65, _D)), const((14, _D)),
                  const((1, _D)), const((_D, _D)), const((_D, _D)),
                  const((_D, _D))],
        out_specs=[rowspec] * 4,
        out_shape=[jax.ShapeDtypeStruct((_T, _D), f32)] * 4,
    )(i1, i2, pos_emb, piece_emb, ln1_0.reshape(1, _D), wq_0, wk_0, wv_0)

    def attn(q, k, v, xres, wo):
        q3 = q.reshape(_B, _SP, _D)
        k3 = k.reshape(_B, _SP, _D)
        v3 = v.reshape(_B, _SP, _D)
        hspec = pl.BlockSpec((1, _SP, 128), lambda b, h2: (b, 0, h2))
        return pl.pallas_call(
            _k_attn,
            grid=(_B, _H // 2),
            in_specs=[hspec, hspec, hspec,
                      pl.BlockSpec((1, _SP, _D), lambda b, h2: (b, 0, 0)),
                      pl.BlockSpec((128, _D), lambda b, h2: (h2, 0))],
            out_specs=pl.BlockSpec((1, _SP, _D), lambda b, h2: (b, 0, 0)),
            out_shape=jax.ShapeDtypeStruct((_B, _SP, _D), f32),
        )(q3, k3, v3, xres.reshape(_B, _SP, _D), wo)

    xa = attn(q0, k0, v0, x0, wo_0).reshape(_T, _D)

    xb, q1, k1, v1 = pl.pallas_call(
        _k_ffn_qkv,
        grid=(_NBLK,),
        in_specs=[rowspec, const((1, _D)), const((_D, _FF)), const((_FF, _D)),
                  const((1, _D)), const((_D, _D)), const((_D, _D)),
                  const((_D, _D))],
        out_specs=[rowspec] * 4,
        out_shape=[jax.ShapeDtypeStruct((_T, _D), f32)] * 4,
    )(xa, ln2_0.reshape(1, _D), ffn_wi_0, ffn_wo_0, ln1_1.reshape(1, _D),
      wq_1, wk_1, wv_1)

    xc = attn(q1, k1, v1, xb, wo_1).reshape(_T, _D)

    h, probs, gate, slot, src8 = pl.pallas_call(
        _k_router,
        grid=(_NBLK,),
        in_specs=[rowspec, const((1, _D)), const((_D, _E)), col1spec],
        out_specs=[rowspec, pl.BlockSpec((_R, _E), lambda i: (i, 0)),
                   col1spec, col1spec, const((_NSLOT, 8))],
        out_shape=[
            jax.ShapeDtypeStruct((_T, _D), f32),
            jax.ShapeDtypeStruct((_T, _E), f32),
            jax.ShapeDtypeStruct((_T, 1), f32),
            jax.ShapeDtypeStruct((_T, 1), jnp.int32),
            jax.ShapeDtypeStruct((_NSLOT, 8), f32),
        ],
        scratch_shapes=[pltpu.VMEM((1, _E), f32)],
    )(xc, ln2_1.reshape(1, _D), router_w, valid)

    src = src8[:, 0].astype(jnp.int32)
    buf = _row_gather(h, src)

    eo = pl.pallas_call(
        _k_expert,
        grid=(_E,),
        in_specs=[pl.BlockSpec((1, _CP, _D), lambda e: (e, 0, 0)),
                  pl.BlockSpec((1, _D, _FF), lambda e: (e, 0, 0)),
                  pl.BlockSpec((1, _FF, _D), lambda e: (e, 0, 0))],
        out_specs=pl.BlockSpec((1, _CP, _D), lambda e: (e, 0, 0)),
        out_shape=jax.ShapeDtypeStruct((_E, _CP, _D), f32),
    )(buf.reshape(_E, _CP, _D), exp_wi, exp_wo)

    slot_pad = jnp.concatenate(
        [slot.reshape(_T), jnp.zeros((_TP - _T,), jnp.int32)])
    y = _row_gather(eo.reshape(_NSLOT, _D), slot_pad)[:_T]

    encoded, out3 = pl.pallas_call(
        _k_final,
        grid=(_NBLK,),
        in_specs=[rowspec, rowspec, col1spec, const((1, _D)),
                  const((_D, 14)), const((1, 14))],
        out_specs=[rowspec, pl.BlockSpec((_R, 14), lambda i: (i, 0))],
        out_shape=[jax.ShapeDtypeStruct((_T, _D), f32),
                   jax.ShapeDtypeStruct((_T, 14), f32)],
    )(xc, y, gate, final_ln.reshape(1, _D), out3_w, out3_b.reshape(1, 14))

    enc4 = encoded.reshape(_B, _SP, 1, _D)
    w1r = out1_w.reshape(_S, _D, _NACT)
    rwr = rew_w.reshape(_S, 1, _D)
    out1_final, values = pl.pallas_call(
        _k_out1,
        grid=(_S,),
        in_specs=[pl.BlockSpec((_B, 1, 1, _D), lambda s: (0, s, 0, 0)),
                  pl.BlockSpec((1, _D, _NACT), lambda s: (s, 0, 0)),
                  pl.BlockSpec((_B, _NACT), lambda s: (0, 0)),
                  pl.BlockSpec((1, _NACT), lambda s: (0, 0)),
                  pl.BlockSpec((1, 1, _D), lambda s: (s, 0, 0)),
                  pl.BlockSpec((1, 1), lambda s: (0, 0))],
        out_specs=[pl.BlockSpec((_B, _NACT), lambda s: (0, 0)),
                   pl.BlockSpec((_B, 1), lambda s: (0, 0))],
        out_shape=[jax.ShapeDtypeStruct((_B, _NACT), f32),
                   jax.ShapeDtypeStruct((_B, 1), f32)],
    )(enc4, w1r, mask, out1_b.reshape(1, _NACT), rwr, rew_b.reshape(1, 1))

    enc3 = encoded.reshape(_B, _SP, _D)
    return (out1_final, out3.reshape(_B, _SP, 14)[:, :_S, :],
            enc3[:, _S - 1, :], values,
            probs.reshape(_B, _SP, _E)[:, :_S, :])


# speedup vs baseline: 1.6162x; 1.3603x over previous
import functools

import jax
import jax.numpy as jnp
from jax import lax
from jax.experimental import pallas as pl
from jax.experimental.pallas import tpu as pltpu
from jax.experimental.pallas import tpu_sc as plsc

_D = 768
_H = 12
_DH = 64
_E = 32
_FF = 2048
_B = 16
_S = 65
_SP = 72
_T = _B * _SP
_C = 65
_CP = 72
_NSLOT = _E * _CP
_TP = 1280
_NACT = 1968
_R = 192
_NBLK = _T // _R

_HI = lax.Precision.HIGHEST


def _dot(a, b, prec=_HI):
    return jax.lax.dot(a, b, preferred_element_type=jnp.float32, precision=prec)


def _bdot(a, b):
    return jax.lax.dot(a.astype(jnp.bfloat16), b.astype(jnp.bfloat16),
                       preferred_element_type=jnp.float32)


def _b16(x):
    return x.astype(jnp.bfloat16).astype(jnp.float32)


def _rmsn(x, w):
    return x * lax.rsqrt(jnp.mean(x * x, axis=-1, keepdims=True) + 1e-6) * w


def _sel_rows(oh, emb):
    oh16 = oh.astype(jnp.bfloat16)
    e0 = emb.astype(jnp.bfloat16)
    r1 = emb - e0.astype(jnp.float32)
    e1 = r1.astype(jnp.bfloat16)
    e2 = (r1 - e1.astype(jnp.float32)).astype(jnp.bfloat16)
    d = lambda w: jax.lax.dot(oh16, w, preferred_element_type=jnp.float32)
    return d(e0) + d(e1) + d(e2)


def _k_embed_qkv(i1_ref, i2_ref, pe_ref, ce_ref, ln_ref, wq_ref, wk_ref,
                 wv_ref, x_ref, q_ref, k_ref, v_ref):
    i1 = i1_ref[...]
    i2 = i2_ref[...]
    oh1 = (lax.broadcasted_iota(jnp.int32, (_R, 65), 1) == i1).astype(jnp.float32)
    oh2 = (lax.broadcasted_iota(jnp.int32, (_R, 14), 1) == i2).astype(jnp.float32)
    x = _sel_rows(oh1, pe_ref[...]) + _sel_rows(oh2, ce_ref[...])
    x_ref[...] = x
    h = _rmsn(x, ln_ref[...])
    q_ref[...] = _bdot(h, wq_ref[...])
    k_ref[...] = _bdot(h, wk_ref[...])
    v_ref[...] = _bdot(h, wv_ref[...])


def _k_attn(q_ref, k_ref, v_ref, out_ref):
    q2 = q_ref[0]
    k2 = k_ref[0]
    v2 = v_ref[0]
    kvalid = lax.broadcasted_iota(jnp.int32, (_SP, _SP), 1) < _S
    outs = []
    for off in (0, _DH):
        qh = q2[:, off:off + _DH]
        kh = k2[:, off:off + _DH]
        vh = v2[:, off:off + _DH]
        s = lax.dot_general(qh.astype(jnp.bfloat16), kh.astype(jnp.bfloat16),
                            (((1,), (1,)), ((), ())),
                            preferred_element_type=jnp.float32) * (1.0 / 8.0)
        s = jnp.where(kvalid, s, -1e30)
        m = jnp.max(s, axis=-1, keepdims=True)
        p = jnp.exp(s - m)
        p = p / jnp.sum(p, axis=-1, keepdims=True)
        outs.append(_bdot(p, vh))
    out_ref[0] = jnp.concatenate(outs, axis=-1)


def _k_ffn_qkv(o_ref, xres_ref, wproj_ref, ln2_ref, wi_ref, wo_ref, ln1_ref,
               wq_ref, wk_ref, wv_ref, x2_ref, q_ref, k_ref, v_ref):
    x = xres_ref[...] + _bdot(o_ref[...], wproj_ref[...])
    h = _rmsn(x, ln2_ref[...])
    f = jnp.maximum(_bdot(h, wi_ref[...]), 0.0)
    x2 = x + _bdot(f, wo_ref[...])
    x2_ref[...] = x2
    h2 = _rmsn(x2, ln1_ref[...])
    q_ref[...] = _bdot(h2, wq_ref[...])
    k_ref[...] = _bdot(h2, wk_ref[...])
    v_ref[...] = _bdot(h2, wv_ref[...])


def _k_router(o_ref, xres_ref, wproj_ref, ln_ref, rw_ref, valid_ref, xc_ref,
              h_ref, probs_ref, gate_ref, slot_ref, src8_ref, carry_ref):
    i = pl.program_id(0)

    @pl.when(i == 0)
    def _():
        carry_ref[...] = jnp.zeros_like(carry_ref)

    x = xres_ref[...] + _bdot(o_ref[...], wproj_ref[...])
    xc_ref[...] = x
    h = _rmsn(x, ln_ref[...])
    h_ref[...] = h
    logits = _bdot(h, rw_ref[...])
    m = jnp.max(logits, axis=-1, keepdims=True)
    ex = jnp.exp(logits - m)
    probs = ex / jnp.sum(ex, axis=-1, keepdims=True)
    probs_ref[...] = probs
    gate = jnp.max(probs, axis=-1, keepdims=True)
    eiota = lax.broadcasted_iota(jnp.int32, (_R, _E), 1)
    idx = jnp.min(jnp.where(probs == gate, eiota, _E), axis=-1, keepdims=True)
    oh = (eiota == idx).astype(jnp.float32) * valid_ref[...]
    tril = (lax.broadcasted_iota(jnp.int32, (_R, _R), 1)
            <= lax.broadcasted_iota(jnp.int32, (_R, _R), 0)).astype(jnp.float32)
    base = carry_ref[...]
    pos = _dot(tril, oh) + base
    carry_ref[...] = base + jnp.sum(oh, axis=0, keepdims=True)
    pos_tok = jnp.sum(pos * oh, axis=-1, keepdims=True)
    keep = (pos_tok >= 1.0) & (pos_tok <= float(_C))
    slot = idx * _CP + (pos_tok.astype(jnp.int32) - 1)
    slot_ref[...] = jnp.where(keep, slot, 0)
    gate_ref[...] = jnp.where(keep, gate, 0.0)
    sdisp = jnp.where(keep, slot, _NSLOT + 7)
    ohts = (lax.broadcasted_iota(jnp.int32, (_R, _NSLOT), 1)
            == sdisp).astype(jnp.float32)
    tvals = (lax.broadcasted_iota(jnp.int32, (_R, 8), 0)
             + i * _R).astype(jnp.float32)
    contrib = lax.dot_general(ohts, tvals, (((0,), (0,)), ((), ())),
                              preferred_element_type=jnp.float32,
                              precision=_HI)

    @pl.when(i == 0)
    def _():
        src8_ref[...] = contrib

    @pl.when(i > 0)
    def _():
        src8_ref[...] += contrib


def _k_expert(b_ref, wi_ref, wo_ref, o_ref):
    x = b_ref[0]
    hmid = jnp.maximum(_bdot(x, wi_ref[0]), 0.0)
    o_ref[0] = _bdot(hmid, wo_ref[0])


def _k_final(x_ref, y_ref, gate_ref, ln_ref, w3_ref, b3_ref, enc_ref, o3_ref):
    xd = x_ref[...] + _b16(gate_ref[...]) * _b16(y_ref[...])
    enc = _rmsn(xd, ln_ref[...])
    enc_ref[...] = enc
    o3_ref[...] = _bdot(enc, w3_ref[...]) + b3_ref[...]


def _k_out1(enc_ref, w1_ref, mask_ref, b1_ref, rw_ref, rb_ref, out_ref, val_ref):
    s = pl.program_id(0)
    e = enc_ref[:, 0, 0, :]
    contrib = lax.dot_general(e.astype(jnp.bfloat16),
                              w1_ref[...].astype(jnp.bfloat16),
                              (((1,), (1,)), ((), ())),
                              preferred_element_type=jnp.float32)
    vcontrib = jnp.sum(_b16(e) * _b16(rw_ref[0]), axis=-1, keepdims=True)

    @pl.when(s == 0)
    def _():
        out_ref[...] = contrib
        val_ref[...] = vcontrib

    @pl.when(s > 0)
    def _():
        out_ref[...] += contrib
        val_ref[...] += vcontrib

    @pl.when(s == _S - 1)
    def _():
        o = out_ref[...] + b1_ref[...]
        out_ref[...] = jnp.where(mask_ref[...] < -1.0, -1e30, o)
        val_ref[...] = jnp.tanh(val_ref[...] + rb_ref[...])


def _row_gather(table, idx):
    v, d = table.shape
    n = idx.shape[0]
    info = plsc.get_sparse_core_info()
    nw = info.num_cores * info.num_subcores
    b_per_w = n // nw
    mesh = plsc.VectorSubcoreMesh(core_axis_name="c", subcore_axis_name="s")

    @functools.partial(
        pl.kernel, mesh=mesh,
        out_type=jax.ShapeDtypeStruct((n, d), jnp.float32),
        scratch_types=[
            pltpu.VMEM((b_per_w,), jnp.int32),
            pltpu.VMEM((b_per_w, d), jnp.float32),
            pltpu.SemaphoreType.DMA,
        ],
    )
    def gk(table_hbm, idx_hbm, out_hbm, idx_v, rows_v, sem):
        wid = lax.axis_index("s") * info.num_cores + lax.axis_index("c")
        base = wid * b_per_w
        pltpu.sync_copy(idx_hbm.at[pl.ds(base, b_per_w)], idx_v)
        pltpu.async_copy(table_hbm.at[idx_v], rows_v, sem).wait()
        pltpu.sync_copy(rows_v, out_hbm.at[pl.ds(base, b_per_w)])

    return gk(table, idx)


def kernel(input1, input2, mask, pos_emb, piece_emb, ln1_0, wq_0, wk_0, wv_0,
           wo_0, ln2_0, ffn_wi_0, ffn_wo_0, ln1_1, wq_1, wk_1, wv_1, wo_1,
           ln2_1, router_w, exp_wi, exp_wo, final_ln, out1_w, out1_b, out3_w,
           out3_b, rew_w, rew_b):
    f32 = jnp.float32
    pad = ((0, 0), (0, _SP - _S))
    i1 = jnp.pad(input1, pad).reshape(_T, 1).astype(jnp.int32)
    i2 = jnp.pad(input2, pad).reshape(_T, 1).astype(jnp.int32)
    valid = (jnp.arange(_T, dtype=jnp.int32) % _SP < _S).astype(f32).reshape(_T, 1)

    rowspec = pl.BlockSpec((_R, _D), lambda i: (i, 0))
    col1spec = pl.BlockSpec((_R, 1), lambda i: (i, 0))

    def const(shape):
        nd = len(shape)
        return pl.BlockSpec(shape, lambda i, _n=nd: (0,) * _n)

    x0, q0, k0, v0 = pl.pallas_call(
        _k_embed_qkv,
        grid=(_NBLK,),
        in_specs=[col1spec, col1spec, const((65, _D)), const((14, _D)),
                  const((1, _D)), const((_D, _D)), const((_D, _D)),
                  const((_D, _D))],
        out_specs=[rowspec] * 4,
        out_shape=[jax.ShapeDtypeStruct((_T, _D), f32)] * 4,
    )(i1, i2, pos_emb, piece_emb, ln1_0.reshape(1, _D), wq_0, wk_0, wv_0)

    def attn(q, k, v):
        q3 = q.reshape(_B, _SP, _D)
        k3 = k.reshape(_B, _SP, _D)
        v3 = v.reshape(_B, _SP, _D)
        hspec = pl.BlockSpec((1, _SP, 128), lambda b, h2: (b, 0, h2))
        return pl.pallas_call(
            _k_attn,
            grid=(_B, _H // 2),
            in_specs=[hspec, hspec, hspec],
            out_specs=pl.BlockSpec((1, _SP, 128), lambda b, h2: (b, 0, h2)),
            out_shape=jax.ShapeDtypeStruct((_B, _SP, _D), f32),
        )(q3, k3, v3).reshape(_T, _D)

    o0 = attn(q0, k0, v0)

    xb, q1, k1, v1 = pl.pallas_call(
        _k_ffn_qkv,
        grid=(_NBLK,),
        in_specs=[rowspec, rowspec, const((_D, _D)), const((1, _D)),
                  const((_D, _FF)), const((_FF, _D)), const((1, _D)),
                  const((_D, _D)), const((_D, _D)), const((_D, _D))],
        out_specs=[rowspec] * 4,
        out_shape=[jax.ShapeDtypeStruct((_T, _D), f32)] * 4,
    )(o0, x0, wo_0, ln2_0.reshape(1, _D), ffn_wi_0, ffn_wo_0,
      ln1_1.reshape(1, _D), wq_1, wk_1, wv_1)

    o1 = attn(q1, k1, v1)

    xc, h, probs, gate, slot, src8 = pl.pallas_call(
        _k_router,
        grid=(_NBLK,),
        in_specs=[rowspec, rowspec, const((_D, _D)), const((1, _D)),
                  const((_D, _E)), col1spec],
        out_specs=[rowspec, rowspec, pl.BlockSpec((_R, _E), lambda i: (i, 0)),
                   col1spec, col1spec, const((_NSLOT, 8))],
        out_shape=[
            jax.ShapeDtypeStruct((_T, _D), f32),
            jax.ShapeDtypeStruct((_T, _D), f32),
            jax.ShapeDtypeStruct((_T, _E), f32),
            jax.ShapeDtypeStruct((_T, 1), f32),
            jax.ShapeDtypeStruct((_T, 1), jnp.int32),
            jax.ShapeDtypeStruct((_NSLOT, 8), f32),
        ],
        scratch_shapes=[pltpu.VMEM((1, _E), f32)],
    )(o1, xb, wo_1, ln2_1.reshape(1, _D), router_w, valid)

    src = src8[:, 0].astype(jnp.int32)
    buf = _row_gather(h, src)

    eo = pl.pallas_call(
        _k_expert,
        grid=(_E,),
        in_specs=[pl.BlockSpec((1, _CP, _D), lambda e: (e, 0, 0)),
                  pl.BlockSpec((1, _D, _FF), lambda e: (e, 0, 0)),
                  pl.BlockSpec((1, _FF, _D), lambda e: (e, 0, 0))],
        out_specs=pl.BlockSpec((1, _CP, _D), lambda e: (e, 0, 0)),
        out_shape=jax.ShapeDtypeStruct((_E, _CP, _D), f32),
    )(buf.reshape(_E, _CP, _D), exp_wi, exp_wo)

    slot_pad = jnp.concatenate(
        [slot.reshape(_T), jnp.zeros((_TP - _T,), jnp.int32)])
    y = _row_gather(eo.reshape(_NSLOT, _D), slot_pad)[:_T]

    encoded, out3 = pl.pallas_call(
        _k_final,
        grid=(_NBLK,),
        in_specs=[rowspec, rowspec, col1spec, const((1, _D)),
                  const((_D, 14)), const((1, 14))],
        out_specs=[rowspec, pl.BlockSpec((_R, 14), lambda i: (i, 0))],
        out_shape=[jax.ShapeDtypeStruct((_T, _D), f32),
                   jax.ShapeDtypeStruct((_T, 14), f32)],
    )(xc, y, gate, final_ln.reshape(1, _D), out3_w, out3_b.reshape(1, 14))

    enc4 = encoded.reshape(_B, _SP, 1, _D)
    w1t = out1_w.T
    rwr = rew_w.reshape(_S, 1, _D)
    out1_final, values = pl.pallas_call(
        _k_out1,
        grid=(_S,),
        in_specs=[pl.BlockSpec((_B, 1, 1, _D), lambda s: (0, s, 0, 0)),
                  pl.BlockSpec((_NACT, _D), lambda s: (0, s)),
                  pl.BlockSpec((_B, _NACT), lambda s: (0, 0)),
                  pl.BlockSpec((1, _NACT), lambda s: (0, 0)),
                  pl.BlockSpec((1, 1, _D), lambda s: (s, 0, 0)),
                  pl.BlockSpec((1, 1), lambda s: (0, 0))],
        out_specs=[pl.BlockSpec((_B, _NACT), lambda s: (0, 0)),
                   pl.BlockSpec((_B, 1), lambda s: (0, 0))],
        out_shape=[jax.ShapeDtypeStruct((_B, _NACT), f32),
                   jax.ShapeDtypeStruct((_B, 1), f32)],
    )(enc4, w1t, mask, out1_b.reshape(1, _NACT), rwr, rew_b.reshape(1, 1))

    enc3 = encoded.reshape(_B, _SP, _D)
    return (out1_final, out3.reshape(_B, _SP, 14)[:, :_S, :],
            enc3[:, _S - 1, :], values,
            probs.reshape(_B, _SP, _E)[:, :_S, :])
